# baseline (speedup 1.0000x reference)
import functools

import jax
import jax.numpy as jnp
from jax import lax
from jax.experimental import pallas as pl
from jax.experimental.pallas import tpu as pltpu
from jax.experimental.pallas import tpu_sc as plsc

N = 10000
E = 320000
D = 128
L = 5
BN_EPS = 1e-5

NC = 2
NS = 16
NW = NC * NS

NN = 10240
EP = 327680
C = 80
PER_W_E = EP // NW
CHUNKS_W = PER_W_E // C
ROWS_T = NN // NS

PER_W_N = NN // NW
EMB_C = 80

BR = 2048

@functools.lru_cache(maxsize=None)
def _sc_kernels():
    mesh = plsc.VectorSubcoreMesh(core_axis_name="c", subcore_axis_name="s",
                                  num_cores=NC, num_subcores=NS)
    emb_gather = functools.partial(
        pl.kernel,
        out_type=jax.ShapeDtypeStruct((NN, D), jnp.float32),
        mesh=mesh,
        scratch_types=[
            pltpu.VMEM((PER_W_N,), jnp.int32),
            pltpu.VMEM((EMB_C, D), jnp.float32),
            pltpu.SemaphoreType.DMA,
        ],
    )(_emb_gather_body)
    seg_sum = functools.partial(
        pl.kernel,
        out_type=jax.ShapeDtypeStruct((2 * NN, D), jnp.float32),
        mesh=mesh,
        scratch_types=[
            pltpu.VMEM((PER_W_E,), jnp.int32),
            pltpu.VMEM((CHUNKS_W, C), jnp.int32),
            pltpu.VMEM((C, D), jnp.float32),
            pltpu.VMEM((C, D), jnp.float32),
            pltpu.VMEM_SHARED((NN, D), jnp.float32),
            pltpu.SemaphoreType.DMA,
            pltpu.SemaphoreType.DMA,
            pltpu.SemaphoreType.DMA,
            pltpu.SemaphoreType.DMA,
        ],
    )(_seg_sum_body)
    return emb_gather, seg_sum


def _emb_gather_body(emb_hbm, xp_hbm, out_hbm, idx_v, rows_v, sem):
    c = lax.axis_index("c")
    s = lax.axis_index("s")
    wid = s * NC + c
    base = pl.multiple_of(wid * PER_W_N, 8)
    pltpu.sync_copy(xp_hbm.at[pl.ds(base, PER_W_N)], idx_v)

    def body(j, carry):
        off = pl.multiple_of(j * EMB_C, 8)
        pltpu.async_copy(emb_hbm.at[idx_v.at[pl.ds(off, EMB_C)]], rows_v, sem).wait()
        pltpu.sync_copy(rows_v, out_hbm.at[pl.ds(base + off, EMB_C)])
        return carry

    lax.fori_loop(0, PER_W_N // EMB_C, body, 0)


def _seg_sum_body(h_hbm, src_hbm, dst2_hbm, out_hbm,
                  src_v, dst_v, buf0_v, buf1_v, acc_sh,
                  sem0, sem1, ssem0, ssem1):
    c = lax.axis_index("c")
    s = lax.axis_index("s")
    wid = s * NC + c

    def zb(t, carry):
        r = t // 8
        k = pl.multiple_of((t % 8) * 16, 16)
        buf0_v[r, pl.ds(k, 16)] = jnp.zeros((16,), jnp.float32)
        buf1_v[r, pl.ds(k, 16)] = jnp.zeros((16,), jnp.float32)
        return carry

    with jax.named_scope("acc_zero"):
        lax.fori_loop(0, C * (D // 16), zb, 0)
        rbase = pl.multiple_of(s * ROWS_T, 8)
        for t in range(ROWS_T // (2 * C)):
            pltpu.sync_copy(buf0_v, acc_sh.at[pl.ds(rbase + 2 * t * C, C)])
            pltpu.sync_copy(buf1_v, acc_sh.at[pl.ds(rbase + (2 * t + 1) * C, C)])
        plsc.subcore_barrier()

    with jax.named_scope("idx_load"):
        ebase = pl.multiple_of(wid * PER_W_E, 8)
        pltpu.sync_copy(src_hbm.at[pl.ds(ebase, PER_W_E)], src_v)
        cbase = pl.multiple_of(wid * CHUNKS_W, 8)
        pltpu.sync_copy(dst2_hbm.at[pl.ds(cbase, CHUNKS_W)], dst_v)

    sc_edges = jax.named_scope("edge_loop")
    sc_edges.__enter__()

    def gather(j, buf, sem):
        off = pl.multiple_of(j * C, 8)
        pltpu.async_copy(h_hbm.at[src_v.at[pl.ds(off, C)]], buf, sem)

    def gwait(buf, sem):
        pltpu.make_async_copy(
            h_hbm.at[src_v.at[pl.ds(pl.multiple_of(0, 8), C)]], buf,
            sem).wait()

    def swait(buf, sem):
        pltpu.make_async_copy(buf, acc_sh.at[dst_v.at[0]], sem).wait()

    gather(0, buf0_v, sem0)
    gather(1, buf1_v, sem1)

    def body(t, carry):
        j0 = 2 * t
        gwait(buf0_v, sem0)
        pltpu.async_copy(buf0_v, acc_sh.at[dst_v.at[j0]], ssem0, add=True)
        gwait(buf1_v, sem1)
        pltpu.async_copy(buf1_v, acc_sh.at[dst_v.at[j0 + 1]], ssem1, add=True)
        swait(buf0_v, ssem0)
        gather(jnp.minimum(j0 + 2, CHUNKS_W - 1), buf0_v, sem0)
        swait(buf1_v, ssem1)
        gather(jnp.minimum(j0 + 3, CHUNKS_W - 1), buf1_v, sem1)
        return carry

    lax.fori_loop(0, CHUNKS_W // 2 - 1, body, 0)
    jl = CHUNKS_W - 2
    gwait(buf0_v, sem0)
    pltpu.async_copy(buf0_v, acc_sh.at[dst_v.at[jl]], ssem0, add=True)
    gwait(buf1_v, sem1)
    pltpu.async_copy(buf1_v, acc_sh.at[dst_v.at[jl + 1]], ssem1, add=True)
    swait(buf0_v, ssem0)
    swait(buf1_v, ssem1)

    plsc.subcore_barrier()
    sc_edges.__exit__(None, None, None)

    with jax.named_scope("acc_out"):
        obase = pl.multiple_of(c * NN + s * ROWS_T, 8)
        pltpu.sync_copy(acc_sh.at[pl.ds(rbase, ROWS_T)],
                        out_hbm.at[pl.ds(obase, ROWS_T)])


def _mlp_body(h_ref, p0_ref, p1_ref, w1_ref, b1_ref, w2_ref, b2_ref, o_ref):
    u = h_ref[...] + p0_ref[...] + p1_ref[...]
    t = jnp.dot(u, w1_ref[...], preferred_element_type=jnp.float32) + b1_ref[...]
    t = jnp.maximum(t, 0.0)
    y = jnp.dot(t, w2_ref[...], preferred_element_type=jnp.float32) + b2_ref[...]
    o_ref[...] = jnp.maximum(y, 0.0)


_mlp = pl.pallas_call(
    _mlp_body,
    grid=(NN // BR,),
    in_specs=[
        pl.BlockSpec((BR, D), lambda i: (i, 0)),
        pl.BlockSpec((BR, D), lambda i: (i, 0)),
        pl.BlockSpec((BR, D), lambda i: (NN // BR + i, 0)),
        pl.BlockSpec((D, D), lambda i: (0, 0)),
        pl.BlockSpec((1, D), lambda i: (0, 0)),
        pl.BlockSpec((D, D), lambda i: (0, 0)),
        pl.BlockSpec((1, D), lambda i: (0, 0)),
    ],
    out_specs=pl.BlockSpec((BR, D), lambda i: (i, 0)),
    out_shape=jax.ShapeDtypeStruct((NN, D), jnp.float32),
)


def _head_body(h_ref, wo1_ref, bo1_ref, wo2_ref, bo2_ref, o_ref):
    rows = lax.broadcasted_iota(jnp.int32, (NN, 1), 0)
    hm = jnp.where(rows < N, h_ref[...], 0.0)
    g = jnp.sum(hm, axis=0, keepdims=True)
    t = jnp.dot(g, wo1_ref[...], preferred_element_type=jnp.float32) + bo1_ref[...]
    t = jnp.maximum(t, 0.0)
    o_ref[...] = jnp.dot(t, wo2_ref[...], preferred_element_type=jnp.float32) + bo2_ref[...]


_head = pl.pallas_call(
    _head_body,
    out_shape=jax.ShapeDtypeStruct((1, D), jnp.float32),
)


def kernel(x, edge_index, batch, emb, W1, b1, g1, be1, W2, b2, g2, be2,
           Wo1, bo1, Wo2, bo2):
    scale = 1.0 / jnp.sqrt(jnp.float32(1.0 + BN_EPS))
    g1s = g1 * scale
    g2s = g2 * scale
    W1f = W1 * g1s[:, None, :]
    b1f = b1 * g1s + be1
    W2f = W2 * g2s[:, None, :]
    b2f = b2 * g2s + be2

    src = edge_index[0].astype(jnp.int32)
    dst = edge_index[1].astype(jnp.int32)
    xp = jnp.concatenate([x.astype(jnp.int32), jnp.zeros((NN - N,), jnp.int32)])
    pad_i = jnp.arange(EP - E, dtype=jnp.int32)
    srcp = jnp.concatenate([src, pad_i % N])
    dstp = jnp.concatenate([dst, N + pad_i % (NN - N)])
    dst2 = dstp.reshape(EP // C, C)

    emb_gather, seg_sum = _sc_kernels()
    h = emb_gather(emb, xp)
    for i in range(L):
        p = seg_sum(h, srcp, dst2)
        h = _mlp(h, p, p, W1f[i], b1f[i].reshape(1, D),
                 W2f[i], b2f[i].reshape(1, D))
    out = _head(h, Wo1, bo1.reshape(1, D), Wo2, bo2.reshape(1, D))
    return out


# speedup vs baseline: 1.2456x; 1.2456x over previous
import functools

import jax
import jax.numpy as jnp
from jax import lax
from jax.experimental import pallas as pl
from jax.experimental.pallas import tpu as pltpu
from jax.experimental.pallas import tpu_sc as plsc

N = 10000
E = 320000
D = 128
L = 5
BN_EPS = 1e-5

NC = 2
NS = 16
NW = NC * NS

NN = 10240
EP = 327680
C = 80
PER_W_E = EP // NW
CHUNKS_W = PER_W_E // C
ROWS_T = NN // NS

PER_W_N = NN // NW
EMB_C = 80

BR = 2048

@functools.lru_cache(maxsize=None)
def _sc_kernels():
    mesh = plsc.VectorSubcoreMesh(core_axis_name="c", subcore_axis_name="s",
                                  num_cores=NC, num_subcores=NS)
    emb_gather = functools.partial(
        pl.kernel,
        out_type=jax.ShapeDtypeStruct((NN, D), jnp.float32),
        mesh=mesh,
        scratch_types=[
            pltpu.VMEM((PER_W_N,), jnp.int32),
            pltpu.VMEM((EMB_C, D), jnp.float32),
            pltpu.SemaphoreType.DMA,
        ],
    )(_emb_gather_body)
    seg_sum = functools.partial(
        pl.kernel,
        out_type=jax.ShapeDtypeStruct((2 * NN, D), jnp.float32),
        mesh=mesh,
        scratch_types=[
            pltpu.VMEM((PER_W_E,), jnp.int32),
            pltpu.VMEM((CHUNKS_W, C), jnp.int32),
            pltpu.VMEM((C, D), jnp.float32),
            pltpu.VMEM((C, D), jnp.float32),
            pltpu.VMEM_SHARED((NN, D), jnp.float32),
            pltpu.SemaphoreType.DMA,
            pltpu.SemaphoreType.DMA,
        ],
    )(_seg_sum_body)
    return emb_gather, seg_sum


def _emb_gather_body(emb_hbm, xp_hbm, out_hbm, idx_v, rows_v, sem):
    c = lax.axis_index("c")
    s = lax.axis_index("s")
    wid = s * NC + c
    base = pl.multiple_of(wid * PER_W_N, 8)
    pltpu.sync_copy(xp_hbm.at[pl.ds(base, PER_W_N)], idx_v)

    def body(j, carry):
        off = pl.multiple_of(j * EMB_C, 8)
        pltpu.async_copy(emb_hbm.at[idx_v.at[pl.ds(off, EMB_C)]], rows_v, sem).wait()
        pltpu.sync_copy(rows_v, out_hbm.at[pl.ds(base + off, EMB_C)])
        return carry

    lax.fori_loop(0, PER_W_N // EMB_C, body, 0)


def _seg_sum_body(h_hbm, src_hbm, dst2_hbm, out_hbm,
                  src_v, dst_v, buf0_v, buf1_v, acc_sh, sem0, sem1):
    c = lax.axis_index("c")
    s = lax.axis_index("s")
    wid = s * NC + c

    def zb(t, carry):
        r = t // 8
        k = pl.multiple_of((t % 8) * 16, 16)
        buf0_v[r, pl.ds(k, 16)] = jnp.zeros((16,), jnp.float32)
        buf1_v[r, pl.ds(k, 16)] = jnp.zeros((16,), jnp.float32)
        return carry

    with jax.named_scope("acc_zero"):
        lax.fori_loop(0, C * (D // 16), zb, 0)
        rbase = pl.multiple_of(s * ROWS_T, 8)
        for t in range(ROWS_T // (2 * C)):
            pltpu.sync_copy(buf0_v, acc_sh.at[pl.ds(rbase + 2 * t * C, C)])
            pltpu.sync_copy(buf1_v, acc_sh.at[pl.ds(rbase + (2 * t + 1) * C, C)])
        plsc.subcore_barrier()

    with jax.named_scope("idx_load"):
        ebase = pl.multiple_of(wid * PER_W_E, 8)
        pltpu.sync_copy(src_hbm.at[pl.ds(ebase, PER_W_E)], src_v)
        cbase = pl.multiple_of(wid * CHUNKS_W, 8)
        pltpu.sync_copy(dst2_hbm.at[pl.ds(cbase, CHUNKS_W)], dst_v)

    sc_edges = jax.named_scope("edge_loop")
    sc_edges.__enter__()

    def gather(j, buf, sem):
        off = pl.multiple_of(j * C, 8)
        pltpu.async_copy(h_hbm.at[src_v.at[pl.ds(off, C)]], buf, sem)

    def gwait(buf, sem):
        pltpu.make_async_copy(
            h_hbm.at[src_v.at[pl.ds(pl.multiple_of(0, 8), C)]], buf,
            sem).wait()

    gather(0, buf0_v, sem0)
    gather(1, buf1_v, sem1)

    def body(t, carry):
        j0 = 2 * t
        gwait(buf0_v, sem0)
        pltpu.sync_copy(buf0_v, acc_sh.at[dst_v.at[j0]], add=True)
        gather(jnp.minimum(j0 + 2, CHUNKS_W - 1), buf0_v, sem0)
        gwait(buf1_v, sem1)
        pltpu.sync_copy(buf1_v, acc_sh.at[dst_v.at[j0 + 1]], add=True)
        gather(jnp.minimum(j0 + 3, CHUNKS_W - 1), buf1_v, sem1)
        return carry

    lax.fori_loop(0, CHUNKS_W // 2, body, 0)
    gwait(buf0_v, sem0)
    gwait(buf1_v, sem1)

    plsc.subcore_barrier()
    sc_edges.__exit__(None, None, None)

    with jax.named_scope("acc_out"):
        obase = pl.multiple_of(c * NN + s * ROWS_T, 8)
        pltpu.sync_copy(acc_sh.at[pl.ds(rbase, ROWS_T)],
                        out_hbm.at[pl.ds(obase, ROWS_T)])


def _mlp_body(h_ref, p0_ref, p1_ref, w1_ref, b1_ref, w2_ref, b2_ref, o_ref):
    u = h_ref[...] + p0_ref[...] + p1_ref[...]
    t = jnp.dot(u, w1_ref[...], preferred_element_type=jnp.float32) + b1_ref[...]
    t = jnp.maximum(t, 0.0)
    y = jnp.dot(t, w2_ref[...], preferred_element_type=jnp.float32) + b2_ref[...]
    o_ref[...] = jnp.maximum(y, 0.0)


_mlp = pl.pallas_call(
    _mlp_body,
    grid=(NN // BR,),
    in_specs=[
        pl.BlockSpec((BR, D), lambda i: (i, 0)),
        pl.BlockSpec((BR, D), lambda i: (i, 0)),
        pl.BlockSpec((BR, D), lambda i: (NN // BR + i, 0)),
        pl.BlockSpec((D, D), lambda i: (0, 0)),
        pl.BlockSpec((1, D), lambda i: (0, 0)),
        pl.BlockSpec((D, D), lambda i: (0, 0)),
        pl.BlockSpec((1, D), lambda i: (0, 0)),
    ],
    out_specs=pl.BlockSpec((BR, D), lambda i: (i, 0)),
    out_shape=jax.ShapeDtypeStruct((NN, D), jnp.float32),
)


def _head_body(h_ref, wo1_ref, bo1_ref, wo2_ref, bo2_ref, o_ref):
    rows = lax.broadcasted_iota(jnp.int32, (NN, 1), 0)
    hm = jnp.where(rows < N, h_ref[...], 0.0)
    g = jnp.sum(hm, axis=0, keepdims=True)
    t = jnp.dot(g, wo1_ref[...], preferred_element_type=jnp.float32) + bo1_ref[...]
    t = jnp.maximum(t, 0.0)
    o_ref[...] = jnp.dot(t, wo2_ref[...], preferred_element_type=jnp.float32) + bo2_ref[...]


_head = pl.pallas_call(
    _head_body,
    out_shape=jax.ShapeDtypeStruct((1, D), jnp.float32),
)


def kernel(x, edge_index, batch, emb, W1, b1, g1, be1, W2, b2, g2, be2,
           Wo1, bo1, Wo2, bo2):
    scale = 1.0 / jnp.sqrt(jnp.float32(1.0 + BN_EPS))
    g1s = g1 * scale
    g2s = g2 * scale
    W1f = W1 * g1s[:, None, :]
    b1f = b1 * g1s + be1
    W2f = W2 * g2s[:, None, :]
    b2f = b2 * g2s + be2

    src = edge_index[0].astype(jnp.int32)
    dst = edge_index[1].astype(jnp.int32)
    xp = jnp.concatenate([x.astype(jnp.int32), jnp.zeros((NN - N,), jnp.int32)])
    pad_i = jnp.arange(EP - E, dtype=jnp.int32)
    srcp = jnp.concatenate([src, pad_i % N])
    dstp = jnp.concatenate([dst, N + pad_i % (NN - N)])
    dst2 = dstp.reshape(EP // C, C)

    emb_gather, seg_sum = _sc_kernels()
    h = emb_gather(emb, xp)
    for i in range(L):
        p = seg_sum(h, srcp, dst2)
        h = _mlp(h, p, p, W1f[i], b1f[i].reshape(1, D),
                 W2f[i], b2f[i].reshape(1, D))
    out = _head(h, Wo1, bo1.reshape(1, D), Wo2, bo2.reshape(1, D))
    return out


# speedup vs baseline: 1.2626x; 1.0136x over previous
import functools

import jax
import jax.numpy as jnp
from jax import lax
from jax.experimental import pallas as pl
from jax.experimental.pallas import tpu as pltpu
from jax.experimental.pallas import tpu_sc as plsc

N = 10000
E = 320000
D = 128
L = 5
BN_EPS = 1e-5

NC = 2
NS = 16
NW = NC * NS

NN = 10240
EP = 327680
C = 80
PER_W_E = EP // NW
CHUNKS_W = PER_W_E // C
ROWS_T = NN // NS

PER_W_N = NN // NW
EMB_C = 80

BR = 2048

@functools.lru_cache(maxsize=None)
def _sc_kernels():
    mesh = plsc.VectorSubcoreMesh(core_axis_name="c", subcore_axis_name="s",
                                  num_cores=NC, num_subcores=NS)
    emb_gather = functools.partial(
        pl.kernel,
        out_type=jax.ShapeDtypeStruct((NN, D), jnp.float32),
        mesh=mesh,
        scratch_types=[
            pltpu.VMEM((PER_W_N,), jnp.int32),
            pltpu.VMEM((EMB_C, D), jnp.float32),
            pltpu.VMEM((EMB_C, D), jnp.float32),
            pltpu.SemaphoreType.DMA,
            pltpu.SemaphoreType.DMA,
        ],
    )(_emb_gather_body)
    seg_sum = functools.partial(
        pl.kernel,
        out_type=jax.ShapeDtypeStruct((2 * NN, D), jnp.float32),
        mesh=mesh,
        scratch_types=[
            pltpu.VMEM((PER_W_E,), jnp.int32),
            pltpu.VMEM((CHUNKS_W, C), jnp.int32),
            pltpu.VMEM((C, D), jnp.float32),
            pltpu.VMEM((C, D), jnp.float32),
            pltpu.VMEM_SHARED((NN, D), jnp.float32),
            pltpu.SemaphoreType.DMA,
            pltpu.SemaphoreType.DMA,
            pltpu.SemaphoreType.DMA,
        ],
    )(_seg_sum_body)
    return emb_gather, seg_sum


def _emb_gather_body(emb_hbm, xp_hbm, out_hbm, idx_v, rows0_v, rows1_v,
                     sem0, sem1):
    c = lax.axis_index("c")
    s = lax.axis_index("s")
    wid = s * NC + c
    base = pl.multiple_of(wid * PER_W_N, 8)
    pltpu.sync_copy(xp_hbm.at[pl.ds(base, PER_W_N)], idx_v)

    NCH = PER_W_N // EMB_C

    def gat(j, buf, sem):
        off = pl.multiple_of(j * EMB_C, 8)
        pltpu.async_copy(emb_hbm.at[idx_v.at[pl.ds(off, EMB_C)]], buf, sem)

    def put(j, buf, sem):
        pltpu.make_async_copy(
            emb_hbm.at[idx_v.at[pl.ds(pl.multiple_of(0, 8), EMB_C)]], buf,
            sem).wait()
        off = pl.multiple_of(j * EMB_C, 8)
        pltpu.sync_copy(buf, out_hbm.at[pl.ds(base + off, EMB_C)])

    gat(0, rows0_v, sem0)
    gat(1, rows1_v, sem1)
    for j in range(NCH):
        buf, sem = (rows0_v, sem0) if j % 2 == 0 else (rows1_v, sem1)
        put(j, buf, sem)
        if j + 2 < NCH:
            gat(j + 2, buf, sem)


def _seg_sum_body(z_hbm, h_hbm, src_hbm, dst2_hbm, out_hbm,
                  src_v, dst_v, buf0_v, buf1_v, acc_sh, sem0, sem1, semz):
    c = lax.axis_index("c")
    s = lax.axis_index("s")
    wid = s * NC + c
    rbase = pl.multiple_of(s * ROWS_T, 8)

    pltpu.async_copy(z_hbm, acc_sh.at[pl.ds(rbase, ROWS_T)], semz)

    with jax.named_scope("idx_load"):
        ebase = pl.multiple_of(wid * PER_W_E, 8)
        pltpu.sync_copy(src_hbm.at[pl.ds(ebase, PER_W_E)], src_v)
        cbase = pl.multiple_of(wid * CHUNKS_W, 8)
        pltpu.sync_copy(dst2_hbm.at[pl.ds(cbase, CHUNKS_W)], dst_v)

    def gather(j, buf, sem):
        off = pl.multiple_of(j * C, 8)
        pltpu.async_copy(h_hbm.at[src_v.at[pl.ds(off, C)]], buf, sem)

    def gwait(buf, sem):
        pltpu.make_async_copy(
            h_hbm.at[src_v.at[pl.ds(pl.multiple_of(0, 8), C)]], buf,
            sem).wait()

    gather(0, buf0_v, sem0)
    gather(1, buf1_v, sem1)

    with jax.named_scope("acc_zero"):
        pltpu.make_async_copy(
            z_hbm, acc_sh.at[pl.ds(rbase, ROWS_T)], semz).wait()
        plsc.subcore_barrier()

    sc_edges = jax.named_scope("edge_loop")
    sc_edges.__enter__()

    def body(t, carry):
        j0 = 2 * t
        gwait(buf0_v, sem0)
        pltpu.sync_copy(buf0_v, acc_sh.at[dst_v.at[j0]], add=True)
        gather(jnp.minimum(j0 + 2, CHUNKS_W - 1), buf0_v, sem0)
        gwait(buf1_v, sem1)
        pltpu.sync_copy(buf1_v, acc_sh.at[dst_v.at[j0 + 1]], add=True)
        gather(jnp.minimum(j0 + 3, CHUNKS_W - 1), buf1_v, sem1)
        return carry

    lax.fori_loop(0, CHUNKS_W // 2, body, 0)
    gwait(buf0_v, sem0)
    gwait(buf1_v, sem1)

    plsc.subcore_barrier()
    sc_edges.__exit__(None, None, None)

    with jax.named_scope("acc_out"):
        obase = pl.multiple_of(c * NN + s * ROWS_T, 8)
        pltpu.sync_copy(acc_sh.at[pl.ds(rbase, ROWS_T)],
                        out_hbm.at[pl.ds(obase, ROWS_T)])


def _mlp_body(h_ref, p0_ref, p1_ref, w1_ref, b1_ref, w2_ref, b2_ref, o_ref):
    u = h_ref[...] + p0_ref[...] + p1_ref[...]
    t = jnp.dot(u, w1_ref[...], preferred_element_type=jnp.float32) + b1_ref[...]
    t = jnp.maximum(t, 0.0)
    y = jnp.dot(t, w2_ref[...], preferred_element_type=jnp.float32) + b2_ref[...]
    o_ref[...] = jnp.maximum(y, 0.0)


_mlp = pl.pallas_call(
    _mlp_body,
    grid=(NN // BR,),
    in_specs=[
        pl.BlockSpec((BR, D), lambda i: (i, 0)),
        pl.BlockSpec((BR, D), lambda i: (i, 0)),
        pl.BlockSpec((BR, D), lambda i: (NN // BR + i, 0)),
        pl.BlockSpec((D, D), lambda i: (0, 0)),
        pl.BlockSpec((1, D), lambda i: (0, 0)),
        pl.BlockSpec((D, D), lambda i: (0, 0)),
        pl.BlockSpec((1, D), lambda i: (0, 0)),
    ],
    out_specs=pl.BlockSpec((BR, D), lambda i: (i, 0)),
    out_shape=jax.ShapeDtypeStruct((NN, D), jnp.float32),
)


def _head_body(h_ref, wo1_ref, bo1_ref, wo2_ref, bo2_ref, o_ref):
    rows = lax.broadcasted_iota(jnp.int32, (NN, 1), 0)
    hm = jnp.where(rows < N, h_ref[...], 0.0)
    g = jnp.sum(hm, axis=0, keepdims=True)
    t = jnp.dot(g, wo1_ref[...], preferred_element_type=jnp.float32) + bo1_ref[...]
    t = jnp.maximum(t, 0.0)
    o_ref[...] = jnp.dot(t, wo2_ref[...], preferred_element_type=jnp.float32) + bo2_ref[...]


_head = pl.pallas_call(
    _head_body,
    out_shape=jax.ShapeDtypeStruct((1, D), jnp.float32),
)


def kernel(x, edge_index, batch, emb, W1, b1, g1, be1, W2, b2, g2, be2,
           Wo1, bo1, Wo2, bo2):
    scale = 1.0 / jnp.sqrt(jnp.float32(1.0 + BN_EPS))
    g1s = g1 * scale
    g2s = g2 * scale
    W1f = W1 * g1s[:, None, :]
    b1f = b1 * g1s + be1
    W2f = W2 * g2s[:, None, :]
    b2f = b2 * g2s + be2

    src = edge_index[0].astype(jnp.int32)
    dst = edge_index[1].astype(jnp.int32)
    xp = jnp.concatenate([x.astype(jnp.int32), jnp.zeros((NN - N,), jnp.int32)])
    pad_i = jnp.arange(EP - E, dtype=jnp.int32)
    srcp = jnp.concatenate([src, pad_i % N])
    dstp = jnp.concatenate([dst, N + pad_i % (NN - N)])
    dst2 = dstp.reshape(EP // C, C)

    emb_gather, seg_sum = _sc_kernels()
    zrows = jnp.zeros((ROWS_T, D), jnp.float32)
    h = emb_gather(emb, xp)
    for i in range(L):
        p = seg_sum(zrows, h, srcp, dst2)
        h = _mlp(h, p, p, W1f[i], b1f[i].reshape(1, D),
                 W2f[i], b2f[i].reshape(1, D))
    out = _head(h, Wo1, bo1.reshape(1, D), Wo2, bo2.reshape(1, D))
    return out
